# baseline (speedup 1.0000x reference)
import jax
import jax.numpy as jnp
from jax.experimental import pallas as pl
from jax.experimental.pallas import tpu as pltpu

N_CATEGORY = 20
N_TYPE_PER_CAT = 10
N_VARIANT_PER_TYPE = 4
N_SPATIAL = 20
D = 320
TOTAL = 1040
VCH = 4
VMAJ = N_CATEGORY // VCH
VROWS = 800 // VCH


def _concat_body(cat_ref, typ_ref, var_ref, spa_ref, out_ref,
                 bcat, btyp, bvar, bspa, obuf, ot,
                 s_cat, s_typ, s_spa, s_out, *s_var):
    c_var = [
        pltpu.make_async_copy(
            var_ref.at[pl.ds(k * VMAJ, VMAJ)],
            bvar.at[pl.ds(k * VMAJ, VMAJ)],
            s_var[k],
        )
        for k in range(VCH)
    ]
    c_typ = pltpu.make_async_copy(typ_ref, btyp, s_typ)
    c_cat = pltpu.make_async_copy(cat_ref, bcat, s_cat)
    c_spa = pltpu.make_async_copy(spa_ref, bspa, s_spa)
    for c in c_var:
        c.start()
    c_typ.start()
    c_cat.start()
    c_spa.start()

    outs = []

    def emit(b, w):
        ot[:, 128 * b:128 * b + w] = jnp.transpose(
            obuf[128 * b:128 * b + w, :])
        o = pltpu.make_async_copy(
            ot.at[:, pl.ds(128 * b, w)],
            out_ref.at[:, pl.ds(128 * b, w)],
            s_out,
        )
        o.start()
        outs.append(o)

    c_cat.wait()
    obuf[0:20] = bcat[...]
    c_typ.wait()
    for i in range(N_CATEGORY):
        obuf[20 + 10 * i:30 + 10 * i] = btyp[:, i, :]
    emit(0, 128)
    c_var[0].wait()
    obuf[220:420] = bvar[0:VMAJ].reshape(VROWS, D)
    emit(1, 128)
    emit(2, 128)
    c_var[1].wait()
    obuf[420:620] = bvar[VMAJ:2 * VMAJ].reshape(VROWS, D)
    emit(3, 128)
    c_var[2].wait()
    obuf[620:820] = bvar[2 * VMAJ:3 * VMAJ].reshape(VROWS, D)
    emit(4, 128)
    emit(5, 128)
    c_spa.wait()
    obuf[1020:1040] = bspa[...]
    c_var[3].wait()
    obuf[820:1020] = bvar[3 * VMAJ:].reshape(VROWS, D)
    emit(6, 128)
    emit(7, 128)
    emit(8, 16)
    for o in outs:
        o.wait()


def kernel(category_codes, type_codes, variant_codes, spatial_codes):
    out_t = pl.pallas_call(
        _concat_body,
        out_shape=jax.ShapeDtypeStruct((D, TOTAL), jnp.float32),
        in_specs=[pl.BlockSpec(memory_space=pl.ANY)] * 4,
        out_specs=pl.BlockSpec(memory_space=pl.ANY),
        scratch_shapes=[
            pltpu.VMEM((N_CATEGORY, D), jnp.float32),
            pltpu.VMEM((N_TYPE_PER_CAT, N_CATEGORY, D), jnp.float32),
            pltpu.VMEM((N_CATEGORY, N_TYPE_PER_CAT, N_VARIANT_PER_TYPE, D),
                       jnp.float32),
            pltpu.VMEM((N_SPATIAL, D), jnp.float32),
            pltpu.VMEM((TOTAL, D), jnp.float32),
            pltpu.VMEM((D, TOTAL), jnp.float32),
        ] + [pltpu.SemaphoreType.DMA] * (4 + VCH),
    )(category_codes, type_codes.transpose(1, 0, 2), variant_codes,
      spatial_codes)
    return out_t.T
